# baseline (speedup 1.0000x reference)
import functools

import jax
import jax.numpy as jnp
from jax import lax
from jax.experimental import pallas as pl
from jax.experimental.pallas import tpu as pltpu
from jax.experimental.pallas import tpu_sc as plsc

_LANES = 16
_N = 256
_TABLE = 1024
_LO = -3.0
_WIDTH = 6.0 / _TABLE
_SCALE = _TABLE / 6.0


@functools.lru_cache(maxsize=None)
def _make_sc_quantize(batch: int):
    try:
        info = plsc.get_sparse_core_info()
        num_cores, num_subcores = info.num_cores, info.num_subcores
    except Exception:
        num_cores, num_subcores = 2, 16
    num_workers = num_cores * num_subcores
    assert batch % (num_workers * _LANES) == 0
    b_per_w = batch // num_workers
    half = b_per_w // 2
    widths = []
    w = _N // 2
    while w >= 1:
        widths.append(w)
        w //= 2

    mesh = plsc.VectorSubcoreMesh(
        core_axis_name="c",
        subcore_axis_name="s",
        num_cores=num_cores,
        num_subcores=num_subcores,
    )

    @functools.partial(
        pl.kernel,
        out_type=jax.ShapeDtypeStruct((batch,), jnp.float32),
        mesh=mesh,
        scratch_types=[
            pltpu.VMEM((b_per_w,), jnp.float32),
            pltpu.VMEM((b_per_w,), jnp.float32),
            pltpu.VMEM((_N + _LANES,), jnp.float32),
            pltpu.VMEM((_N,), jnp.float32),
            pltpu.VMEM((_TABLE,), jnp.float32),
            pltpu.VMEM((_TABLE,), jnp.float32),
            pltpu.VMEM((_TABLE,), jnp.float32),
            pltpu.SemaphoreType.DMA,
            pltpu.SemaphoreType.DMA,
            pltpu.SemaphoreType.DMA,
        ],
        compiler_params=pltpu.CompilerParams(needs_layout_passes=False),
    )
    def quantize(
        x_hbm, centers_hbm, out_hbm,
        x_v, o_v, cen_v, bnd_v, bval_v, clo_v, chi_v, sem_in, sem_out, sem_c,
    ):
        wid = lax.axis_index("s") * num_cores + lax.axis_index("c")
        base = wid * b_per_w
        in_copy = pltpu.async_copy(x_hbm.at[pl.ds(base, b_per_w)], x_v, sem_in)
        c_copy = pltpu.async_copy(centers_hbm, cen_v.at[pl.ds(0, _N)], sem_c)

        lane = lax.iota(jnp.int32, _LANES)
        c_copy.wait()

        @plsc.parallel_loop(0, _N // _LANES, 1, unroll=4)
        def _(j):
            lo = cen_v[pl.ds(j * _LANES, _LANES)]
            hi = plsc.load_gather(cen_v, [lane + (j * _LANES + 1)])
            mid = (lo + hi) * 0.5
            mid = jnp.where(
                lane + j * _LANES == _N - 1, jnp.full((_LANES,), jnp.inf, jnp.float32), mid
            )
            bnd_v[pl.ds(j * _LANES, _LANES)] = mid

        @plsc.parallel_loop(0, _TABLE // _LANES, 1, unroll=4)
        def _(j):
            g = (j * _LANES + lane).astype(jnp.float32) * _WIDTH + _LO
            pos = jnp.zeros((_LANES,), jnp.int32)
            for w in widths:
                mv = plsc.load_gather(bnd_v, [pos + (w - 1)])
                pos = jnp.where(mv < g, pos + w, pos)
            bval_v[pl.ds(j * _LANES, _LANES)] = plsc.load_gather(bnd_v, [pos])
            clo_v[pl.ds(j * _LANES, _LANES)] = plsc.load_gather(cen_v, [pos])
            chi_v[pl.ds(j * _LANES, _LANES)] = plsc.load_gather(cen_v, [pos + 1])

        in_copy.wait()

        @plsc.parallel_loop(0, half // _LANES, 1, unroll=16)
        def _(i):
            x = x_v[pl.ds(i * _LANES, _LANES)]
            t = jnp.clip(((x - _LO) * _SCALE).astype(jnp.int32), 0, _TABLE - 1)
            bv = plsc.load_gather(bval_v, [t])
            clo = plsc.load_gather(clo_v, [t])
            chi = plsc.load_gather(chi_v, [t])
            o_v[pl.ds(i * _LANES, _LANES)] = jnp.where(bv < x, chi, clo)

        out_copy1 = pltpu.async_copy(
            o_v.at[pl.ds(0, half)], out_hbm.at[pl.ds(base, half)], sem_out
        )

        @plsc.parallel_loop(half // _LANES, b_per_w // _LANES, 1, unroll=16)
        def _(i):
            x = x_v[pl.ds(i * _LANES, _LANES)]
            t = jnp.clip(((x - _LO) * _SCALE).astype(jnp.int32), 0, _TABLE - 1)
            bv = plsc.load_gather(bval_v, [t])
            clo = plsc.load_gather(clo_v, [t])
            chi = plsc.load_gather(chi_v, [t])
            o_v[pl.ds(i * _LANES, _LANES)] = jnp.where(bv < x, chi, clo)

        out_copy2 = pltpu.async_copy(
            o_v.at[pl.ds(half, half)], out_hbm.at[pl.ds(base + half, half)], sem_out
        )
        out_copy1.wait()
        out_copy2.wait()

    return quantize


def kernel(sample, centers):
    x = sample.reshape(-1)
    c = centers.reshape(-1)
    out = _make_sc_quantize(x.shape[0])(x, c)
    return out.reshape(-1, 1)


# speedup vs baseline: 1.0110x; 1.0110x over previous
import functools

import jax
import jax.numpy as jnp
from jax import lax
from jax.experimental import pallas as pl
from jax.experimental.pallas import tpu as pltpu
from jax.experimental.pallas import tpu_sc as plsc

_LANES = 16
_N = 256
_TABLE = 1024
_LO = -3.0
_WIDTH = 6.0 / _TABLE
_SCALE = _TABLE / 6.0


@functools.lru_cache(maxsize=None)
def _make_sc_quantize(batch: int):
    try:
        info = plsc.get_sparse_core_info()
        num_cores, num_subcores = info.num_cores, info.num_subcores
    except Exception:
        num_cores, num_subcores = 2, 16
    num_workers = num_cores * num_subcores
    assert batch % (num_workers * _LANES) == 0
    b_per_w = batch // num_workers
    half = b_per_w // 2
    widths = []
    w = _N // 2
    while w >= 1:
        widths.append(w)
        w //= 2

    mesh = plsc.VectorSubcoreMesh(
        core_axis_name="c",
        subcore_axis_name="s",
        num_cores=num_cores,
        num_subcores=num_subcores,
    )

    @functools.partial(
        pl.kernel,
        out_type=jax.ShapeDtypeStruct((batch,), jnp.float32),
        mesh=mesh,
        scratch_types=[
            pltpu.VMEM((b_per_w,), jnp.float32),
            pltpu.VMEM((b_per_w,), jnp.float32),
            pltpu.VMEM((_N + _LANES,), jnp.float32),
            pltpu.VMEM((_N,), jnp.float32),
            pltpu.VMEM((_TABLE,), jnp.float32),
            pltpu.VMEM((_TABLE,), jnp.float32),
            pltpu.VMEM((_TABLE,), jnp.float32),
            pltpu.SemaphoreType.DMA,
            pltpu.SemaphoreType.DMA,
            pltpu.SemaphoreType.DMA,
        ],
        compiler_params=pltpu.CompilerParams(needs_layout_passes=False),
    )
    def quantize(
        x_hbm, centers_hbm, out_hbm,
        x_v, o_v, cen_v, bnd_v, bval_v, clo_v, chi_v, sem_in, sem_out, sem_c,
    ):
        wid = lax.axis_index("s") * num_cores + lax.axis_index("c")
        base = wid * b_per_w
        in_copy = pltpu.async_copy(x_hbm.at[pl.ds(base, b_per_w)], x_v, sem_in)
        c_copy = pltpu.async_copy(centers_hbm, cen_v.at[pl.ds(0, _N)], sem_c)

        lane = lax.iota(jnp.int32, _LANES)
        c_copy.wait()

        @plsc.parallel_loop(0, _N // _LANES, 1, unroll=4)
        def _(j):
            lo = cen_v[pl.ds(j * _LANES, _LANES)]
            hi = plsc.load_gather(cen_v, [lane + (j * _LANES + 1)])
            mid = (lo + hi) * 0.5
            mid = jnp.where(
                lane + j * _LANES == _N - 1, jnp.full((_LANES,), jnp.inf, jnp.float32), mid
            )
            bnd_v[pl.ds(j * _LANES, _LANES)] = mid

        @plsc.parallel_loop(0, _TABLE // _LANES, 1, unroll=8)
        def _(j):
            g = (j * _LANES + lane).astype(jnp.float32) * _WIDTH + _LO
            pos = jnp.zeros((_LANES,), jnp.int32)
            for w in widths:
                mv = plsc.load_gather(bnd_v, [pos + (w - 1)])
                pos = jnp.where(mv < g, pos + w, pos)
            bval_v[pl.ds(j * _LANES, _LANES)] = plsc.load_gather(bnd_v, [pos])
            clo_v[pl.ds(j * _LANES, _LANES)] = plsc.load_gather(cen_v, [pos])
            chi_v[pl.ds(j * _LANES, _LANES)] = plsc.load_gather(cen_v, [pos + 1])

        in_copy.wait()

        @plsc.parallel_loop(0, b_per_w // _LANES, 1, unroll=8)
        def _(i):
            x = x_v[pl.ds(i * _LANES, _LANES)]
            t = jnp.clip(((x - _LO) * _SCALE).astype(jnp.int32), 0, _TABLE - 1)
            bv = plsc.load_gather(bval_v, [t])
            clo = plsc.load_gather(clo_v, [t])
            chi = plsc.load_gather(chi_v, [t])
            o_v[pl.ds(i * _LANES, _LANES)] = jnp.where(bv < x, chi, clo)

        pltpu.sync_copy(o_v, out_hbm.at[pl.ds(base, b_per_w)])

    return quantize


def kernel(sample, centers):
    x = sample.reshape(-1)
    c = centers.reshape(-1)
    out = _make_sc_quantize(x.shape[0])(x, c)
    return out.reshape(-1, 1)


# speedup vs baseline: 1.0165x; 1.0055x over previous
import functools

import jax
import jax.numpy as jnp
from jax import lax
from jax.experimental import pallas as pl
from jax.experimental.pallas import tpu as pltpu
from jax.experimental.pallas import tpu_sc as plsc

_LANES = 16
_N = 256
_TABLE = 1024
_LO = -3.0
_WIDTH = 6.0 / _TABLE
_SCALE = _TABLE / 6.0


@functools.lru_cache(maxsize=None)
def _make_sc_quantize(batch: int):
    try:
        info = plsc.get_sparse_core_info()
        num_cores, num_subcores = info.num_cores, info.num_subcores
    except Exception:
        num_cores, num_subcores = 2, 16
    num_workers = num_cores * num_subcores
    assert batch % (num_workers * _LANES) == 0
    b_per_w = batch // num_workers
    widths = []
    w = _N // 2
    while w >= 1:
        widths.append(w)
        w //= 2

    mesh = plsc.VectorSubcoreMesh(
        core_axis_name="c",
        subcore_axis_name="s",
        num_cores=num_cores,
        num_subcores=num_subcores,
    )

    @functools.partial(
        pl.kernel,
        out_type=jax.ShapeDtypeStruct((batch,), jnp.float32),
        mesh=mesh,
        scratch_types=[
            pltpu.VMEM((b_per_w,), jnp.float32),
            pltpu.VMEM((b_per_w,), jnp.float32),
            pltpu.VMEM((_N + _LANES,), jnp.float32),
            pltpu.VMEM((_N,), jnp.float32),
            pltpu.VMEM((_TABLE,), jnp.float32),
            pltpu.VMEM((_TABLE,), jnp.float32),
            pltpu.VMEM((_TABLE,), jnp.float32),
            pltpu.SemaphoreType.DMA,
            pltpu.SemaphoreType.DMA,
        ],
        compiler_params=pltpu.CompilerParams(needs_layout_passes=False),
    )
    def quantize(
        x_hbm, centers_hbm, out_hbm,
        x_v, o_v, cen_v, bnd_v, bval_v, clo_v, chi_v, sem_in, sem_c,
    ):
        wid = lax.axis_index("s") * num_cores + lax.axis_index("c")
        base = wid * b_per_w
        in_copy = pltpu.async_copy(x_hbm.at[pl.ds(base, b_per_w)], x_v, sem_in)
        c_copy = pltpu.async_copy(centers_hbm, cen_v.at[pl.ds(0, _N)], sem_c)

        lane = lax.iota(jnp.int32, _LANES)
        c_copy.wait()

        @plsc.parallel_loop(0, _N // _LANES, 1, unroll=4)
        def _(j):
            lo = cen_v[pl.ds(j * _LANES, _LANES)]
            hi = plsc.load_gather(cen_v, [lane + (j * _LANES + 1)])
            mid = (lo + hi) * 0.5
            mid = jnp.where(
                lane + j * _LANES == _N - 1, jnp.full((_LANES,), jnp.inf, jnp.float32), mid
            )
            bnd_v[pl.ds(j * _LANES, _LANES)] = mid

        @plsc.parallel_loop(0, _TABLE // _LANES, 1, unroll=8)
        def _(j):
            g = (j * _LANES + lane).astype(jnp.float32) * _WIDTH + _LO
            pos = jnp.zeros((_LANES,), jnp.int32)
            for w in widths:
                mv = plsc.load_gather(bnd_v, [pos + (w - 1)])
                pos = jnp.where(mv < g, pos + w, pos)
            bval_v[pl.ds(j * _LANES, _LANES)] = plsc.load_gather(bnd_v, [pos])
            clo_v[pl.ds(j * _LANES, _LANES)] = plsc.load_gather(cen_v, [pos])
            chi_v[pl.ds(j * _LANES, _LANES)] = plsc.load_gather(cen_v, [pos + 1])

        in_copy.wait()

        @plsc.parallel_loop(0, b_per_w // _LANES, 1, unroll=8)
        def _(i):
            x = x_v[pl.ds(i * _LANES, _LANES)]
            t = jnp.clip(((x - _LO) * _SCALE).astype(jnp.int32), 0, _TABLE - 1)
            bv = plsc.load_gather(bval_v, [t])
            clo = plsc.load_gather(clo_v, [t])
            chi = plsc.load_gather(chi_v, [t])
            o_v[pl.ds(i * _LANES, _LANES)] = jnp.where(bv < x, chi, clo)

        pltpu.sync_copy(o_v, out_hbm.at[pl.ds(base, b_per_w)])

    return quantize


def kernel(sample, centers):
    x = sample.reshape(-1)
    c = centers.reshape(-1)
    out = _make_sc_quantize(x.shape[0])(x, c)
    return out.reshape(-1, 1)
